# Initial kernel scaffold; baseline (speedup 1.0000x reference)
import jax
import jax.numpy as jnp
from jax.experimental import pallas as pl


def kernel(target_x, hist_x, hist_card_local_idx, target_card_local_idx, card_dense_feats, W1, b1, W2, b2, W3, b3, W4, b4, W5, b5, W6, b6):
    raise NotImplementedError("write your pallas kernel here")


# speedup vs baseline: 2.7896x; 2.7896x over previous
import jax
import jax.numpy as jnp
from jax import lax
from jax.experimental import pallas as pl
from jax.experimental.pallas import tpu as pltpu
from jax.experimental.pallas import tpu_sc as plsc

F = 128
D = 64
CF = 5
CH = 128


def _enc_body(x_ref, w1_ref, b1_ref, w2_ref, b2_ref, o_ref):
    h = jnp.dot(x_ref[...], w1_ref[...], preferred_element_type=jnp.float32)
    h = jnp.maximum(h + b1_ref[...], 0.0)
    h = jnp.dot(h, w2_ref[...], preferred_element_type=jnp.float32)
    o_ref[...] = jnp.maximum(h + b2_ref[...], 0.0)


def _encode(x, w1, b1, w2, b2, blk):
    n = x.shape[0]
    return pl.pallas_call(
        _enc_body,
        grid=(n // blk,),
        in_specs=[
            pl.BlockSpec((blk, F), lambda i: (i, 0)),
            pl.BlockSpec((F, D), lambda i: (0, 0)),
            pl.BlockSpec((1, D), lambda i: (0, 0)),
            pl.BlockSpec((D, D), lambda i: (0, 0)),
            pl.BlockSpec((1, D), lambda i: (0, 0)),
        ],
        out_specs=pl.BlockSpec((blk, D), lambda i: (i, 0)),
        out_shape=jax.ShapeDtypeStruct((n, D), jnp.float32),
    )(x, w1, b1.reshape(1, D), w2, b2.reshape(1, D))


def _card_body(ps_ref, pc_ref, dense_ref, w3a_ref, w3b_ref, b3_ref,
               w4_ref, b4_ref, o_ref):
    s = ps_ref[0] + ps_ref[1]
    cnt = pc_ref[0, :, 0:1] + pc_ref[1, :, 0:1]
    agg = s / jnp.maximum(cnt, 1.0)
    ch = agg @ w3b_ref[...] + b3_ref[...]
    for i in range(CF):
        ch = ch + dense_ref[:, i:i + 1] * w3a_ref[i:i + 1, :]
    ch = jnp.maximum(ch, 0.0)
    h = jnp.dot(ch, w4_ref[...], preferred_element_type=jnp.float32)
    o_ref[...] = jnp.maximum(h + b4_ref[...], 0.0)


def _card_mlp(psum, pcnt, dense, w3a, w3b, b3, w4, b4):
    u = dense.shape[0]
    return pl.pallas_call(
        _card_body,
        out_shape=jax.ShapeDtypeStruct((u, D), jnp.float32),
    )(psum, pcnt, dense, w3a, w3b, b3.reshape(1, D), w4, b4.reshape(1, D))


def _head_body(th_ref, tch_ref, w5a_ref, w5b_ref, b5_ref, w6_ref, b6_ref,
               o_ref):
    h = (jnp.dot(th_ref[...], w5a_ref[...], preferred_element_type=jnp.float32)
         + jnp.dot(tch_ref[...], w5b_ref[...],
                   preferred_element_type=jnp.float32))
    h = jnp.maximum(h + b5_ref[...], 0.0)
    o_ref[...] = jnp.dot(h, w6_ref[...],
                         preferred_element_type=jnp.float32) + b6_ref[...]


def _head(th, tch, w5a, w5b, b5, w6, b6, blk):
    n = th.shape[0]
    return pl.pallas_call(
        _head_body,
        grid=(n // blk,),
        in_specs=[
            pl.BlockSpec((blk, D), lambda i: (i, 0)),
            pl.BlockSpec((blk, D), lambda i: (i, 0)),
            pl.BlockSpec((D, D), lambda i: (0, 0)),
            pl.BlockSpec((D, D), lambda i: (0, 0)),
            pl.BlockSpec((1, D), lambda i: (0, 0)),
            pl.BlockSpec((D, 1), lambda i: (0, 0)),
            pl.BlockSpec((1, 1), lambda i: (0, 0)),
        ],
        out_specs=pl.BlockSpec((blk, 1), lambda i: (i, 0)),
        out_shape=jax.ShapeDtypeStruct((n, 1), jnp.float32),
    )(th, tch, w5a, w5b, b5.reshape(1, D), w6, b6.reshape(1, 1))


_VMESH = plsc.VectorSubcoreMesh(core_axis_name="core", subcore_axis_name="subcore")
_SC_PARAMS = pltpu.CompilerParams(use_tc_tiling_on_sc=False)


def _sc_segment_sum(hh, idx2d, zsum, zcnt, u):
    h = hh.shape[0]
    rows_per_sub = u // 16

    @pl.kernel(
        out_type=(jax.ShapeDtypeStruct((2, u, D), jnp.float32),
                  jax.ShapeDtypeStruct((2, u, 16), jnp.float32)),
        mesh=_VMESH,
        compiler_params=_SC_PARAMS,
        scratch_types=[
            pltpu.VMEM_SHARED((u, D), jnp.float32),
            pltpu.VMEM_SHARED((u, 16), jnp.float32),
            pltpu.VMEM((CH, 16), jnp.float32),
        ],
    )
    def sc_kernel(hh_hbm, idx_hbm, zsum_hbm, zcnt_hbm, osum_hbm, ocnt_hbm,
                  acc_sum, acc_cnt, ones_v):
        cid = lax.axis_index("core")
        sid = lax.axis_index("subcore")

        @pl.loop(0, CH)
        def _(i):
            ones_v.at[pl.ds(i, 1), :][...] = jnp.ones((1, 16), jnp.float32)

        sl = pl.ds(sid * rows_per_sub, rows_per_sub)
        pltpu.sync_copy(zsum_hbm.at[sl], acc_sum.at[sl])
        pltpu.sync_copy(zcnt_hbm.at[sl], acc_cnt.at[sl])
        plsc.subcore_barrier()

        def body(x_vmem, i_vmem):
            pltpu.sync_copy(x_vmem, acc_sum.at[i_vmem.at[0]], add=True)
            pltpu.sync_copy(ones_v, acc_cnt.at[i_vmem.at[0]], add=True)

        pltpu.emit_pipeline(
            body,
            grid=(h // CH,),
            in_specs=[
                pl.BlockSpec((CH, D), lambda i: (i, 0)),
                pl.BlockSpec((1, CH), lambda i: (0, i)),
            ],
            out_specs=[],
            core_axis_name=("core", "subcore"),
            dimension_semantics=(pltpu.PARALLEL,),
        )(hh_hbm, idx_hbm)

        plsc.subcore_barrier()
        pltpu.sync_copy(acc_sum.at[sl], osum_hbm.at[cid, sl])
        pltpu.sync_copy(acc_cnt.at[sl], ocnt_hbm.at[cid, sl])

    return sc_kernel(hh, idx2d, zsum, zcnt)


def _sc_gather(table, idx2d):
    n = idx2d.shape[1]

    @pl.kernel(
        out_type=jax.ShapeDtypeStruct((n, D), jnp.float32),
        mesh=_VMESH,
        compiler_params=_SC_PARAMS,
    )
    def sc_kernel(tab_hbm, i_hbm, o_hbm):
        def body(i_vmem, o_vmem):
            pltpu.sync_copy(tab_hbm.at[i_vmem.at[0]], o_vmem)

        pltpu.emit_pipeline(
            body,
            grid=(n // CH,),
            in_specs=[pl.BlockSpec((1, CH), lambda i: (0, i))],
            out_specs=[pl.BlockSpec((CH, D), lambda i: (i, 0))],
            core_axis_name=("core", "subcore"),
            dimension_semantics=(pltpu.PARALLEL,),
        )(i_hbm, o_hbm)

    return sc_kernel(table, idx2d)


def kernel(target_x, hist_x, hist_card_local_idx, target_card_local_idx,
           card_dense_feats, W1, b1, W2, b2, W3, b3, W4, b4, W5, b5, W6, b6):
    b = target_x.shape[0]
    u = card_dense_feats.shape[0]

    hist_h = _encode(hist_x, W1, b1, W2, b2, blk=2560)
    target_h = _encode(target_x, W1, b1, W2, b2, blk=2000)

    up = ((u + 127) // 128) * 128
    zsum = jnp.zeros((up, D), jnp.float32)
    zcnt = jnp.zeros((up, 16), jnp.float32)
    psum, pcnt = _sc_segment_sum(
        hist_h, hist_card_local_idx.reshape(1, -1), zsum, zcnt, up)

    dense_p = jnp.pad(card_dense_feats, ((0, up - u), (0, 0)))
    card_h = _card_mlp(psum, pcnt, dense_p,
                       W3[:CF], W3[CF:], b3, W4, b4)

    bp = ((b + CH - 1) // CH) * CH
    tidx = jnp.pad(target_card_local_idx, (0, bp - b)).reshape(1, bp)
    tch = _sc_gather(card_h, tidx)

    logits = _head(target_h, tch[:b], W5[:D], W5[D:], b5, W6, b6, blk=2000)
    return logits.reshape(b)


# speedup vs baseline: 2.9987x; 1.0750x over previous
import jax
import jax.numpy as jnp
from jax import lax
from jax.experimental import pallas as pl
from jax.experimental.pallas import tpu as pltpu
from jax.experimental.pallas import tpu_sc as plsc

F = 128
D = 64
CF = 5
CH = 128


def _mm(a, b):
    return jnp.dot(a.astype(jnp.bfloat16), b.astype(jnp.bfloat16),
                   preferred_element_type=jnp.float32)


def _enc_body(x_ref, w1_ref, b1_ref, w2_ref, b2_ref, o_ref):
    h = jnp.maximum(_mm(x_ref[...], w1_ref[...]) + b1_ref[...], 0.0)
    o_ref[...] = jnp.maximum(_mm(h, w2_ref[...]) + b2_ref[...], 0.0)


def _encode(x, w1, b1, w2, b2, blk):
    n = x.shape[0]
    return pl.pallas_call(
        _enc_body,
        grid=(n // blk,),
        in_specs=[
            pl.BlockSpec((blk, F), lambda i: (i, 0)),
            pl.BlockSpec((F, D), lambda i: (0, 0)),
            pl.BlockSpec((1, D), lambda i: (0, 0)),
            pl.BlockSpec((D, D), lambda i: (0, 0)),
            pl.BlockSpec((1, D), lambda i: (0, 0)),
        ],
        out_specs=pl.BlockSpec((blk, D), lambda i: (i, 0)),
        out_shape=jax.ShapeDtypeStruct((n, D), jnp.float32),
    )(x, w1, b1.reshape(1, D), w2, b2.reshape(1, D))


def _card_body(ps_ref, pc_ref, dense_ref, w3a_ref, w3b_ref, b3_ref,
               w4_ref, b4_ref, o_ref):
    s = ps_ref[0] + ps_ref[1]
    cnt = pc_ref[0, :, 0:1] + pc_ref[1, :, 0:1]
    agg = s / jnp.maximum(cnt, 1.0)
    ch = _mm(agg, w3b_ref[...]) + b3_ref[...]
    for i in range(CF):
        ch = ch + dense_ref[:, i:i + 1] * w3a_ref[i:i + 1, :]
    ch = jnp.maximum(ch, 0.0)
    h = jnp.maximum(_mm(ch, w4_ref[...]) + b4_ref[...], 0.0)
    o_ref[...] = jnp.concatenate([h, jnp.zeros_like(h)], axis=1)


def _card_mlp(psum, pcnt, dense, w3a, w3b, b3, w4, b4):
    u = dense.shape[0]
    return pl.pallas_call(
        _card_body,
        out_shape=jax.ShapeDtypeStruct((u, 2 * D), jnp.float32),
    )(psum, pcnt, dense, w3a, w3b, b3.reshape(1, D), w4, b4.reshape(1, D))


def _head_body(x_ref, tch_ref, w1_ref, b1_ref, w2_ref, b2_ref,
               w5a_ref, w5b_ref, b5_ref, w6_ref, b6_ref, o_ref):
    th = jnp.maximum(_mm(x_ref[...], w1_ref[...]) + b1_ref[...], 0.0)
    th = jnp.maximum(_mm(th, w2_ref[...]) + b2_ref[...], 0.0)
    h = _mm(th, w5a_ref[...]) + _mm(tch_ref[...], w5b_ref[...])
    h = jnp.maximum(h + b5_ref[...], 0.0)
    o_ref[...] = _mm(h, w6_ref[...]) + b6_ref[...]


def _head(target_x, tch, w1, b1, w2, b2, w5a, w5bp, b5, w6, b6, blk):
    n = target_x.shape[0]
    return pl.pallas_call(
        _head_body,
        grid=(n // blk,),
        in_specs=[
            pl.BlockSpec((blk, F), lambda i: (i, 0)),
            pl.BlockSpec((blk, 2 * D), lambda i: (i, 0)),
            pl.BlockSpec((F, D), lambda i: (0, 0)),
            pl.BlockSpec((1, D), lambda i: (0, 0)),
            pl.BlockSpec((D, D), lambda i: (0, 0)),
            pl.BlockSpec((1, D), lambda i: (0, 0)),
            pl.BlockSpec((D, D), lambda i: (0, 0)),
            pl.BlockSpec((2 * D, D), lambda i: (0, 0)),
            pl.BlockSpec((1, D), lambda i: (0, 0)),
            pl.BlockSpec((D, 1), lambda i: (0, 0)),
            pl.BlockSpec((1, 1), lambda i: (0, 0)),
        ],
        out_specs=pl.BlockSpec((blk, 1), lambda i: (i, 0)),
        out_shape=jax.ShapeDtypeStruct((n, 1), jnp.float32),
    )(target_x, tch, w1, b1.reshape(1, D), w2, b2.reshape(1, D),
      w5a, w5bp, b5.reshape(1, D), w6, b6.reshape(1, 1))


_VMESH = plsc.VectorSubcoreMesh(core_axis_name="core", subcore_axis_name="subcore")
_SC_PARAMS = pltpu.CompilerParams(use_tc_tiling_on_sc=False)


def _sc_segment_sum(hh, idx2d, zsum, zcnt, u):
    h = hh.shape[0]
    rows_per_sub = u // 16

    @pl.kernel(
        out_type=(jax.ShapeDtypeStruct((2, u, D), jnp.float32),
                  jax.ShapeDtypeStruct((2, u, 16), jnp.float32)),
        mesh=_VMESH,
        compiler_params=_SC_PARAMS,
        scratch_types=[
            pltpu.VMEM_SHARED((u, D), jnp.float32),
            pltpu.VMEM_SHARED((u, 16), jnp.float32),
            pltpu.VMEM((CH, 16), jnp.float32),
        ],
    )
    def sc_kernel(hh_hbm, idx_hbm, zsum_hbm, zcnt_hbm, osum_hbm, ocnt_hbm,
                  acc_sum, acc_cnt, ones_v):
        cid = lax.axis_index("core")
        sid = lax.axis_index("subcore")

        @pl.loop(0, CH)
        def _(i):
            ones_v.at[pl.ds(i, 1), :][...] = jnp.ones((1, 16), jnp.float32)

        sl = pl.ds(sid * rows_per_sub, rows_per_sub)
        pltpu.sync_copy(zsum_hbm.at[sl], acc_sum.at[sl])
        pltpu.sync_copy(zcnt_hbm.at[sl], acc_cnt.at[sl])
        plsc.subcore_barrier()

        def body(x_vmem, i_vmem):
            pltpu.sync_copy(x_vmem, acc_sum.at[i_vmem.at[0]], add=True)
            pltpu.sync_copy(ones_v, acc_cnt.at[i_vmem.at[0]], add=True)

        pltpu.emit_pipeline(
            body,
            grid=(h // CH,),
            in_specs=[
                pl.BlockSpec((CH, D), lambda i: (i, 0)),
                pl.BlockSpec((1, CH), lambda i: (0, i)),
            ],
            out_specs=[],
            core_axis_name=("core", "subcore"),
            dimension_semantics=(pltpu.PARALLEL,),
        )(hh_hbm, idx_hbm)

        plsc.subcore_barrier()
        pltpu.sync_copy(acc_sum.at[sl], osum_hbm.at[cid, sl])
        pltpu.sync_copy(acc_cnt.at[sl], ocnt_hbm.at[cid, sl])

    return sc_kernel(hh, idx2d, zsum, zcnt)


def _sc_gather(table, idx2d):
    n = idx2d.shape[1]

    @pl.kernel(
        out_type=jax.ShapeDtypeStruct((n, 2 * D), jnp.float32),
        mesh=_VMESH,
    )
    def sc_kernel(tab_hbm, i_hbm, o_hbm):
        def body(i_vmem, o_vmem):
            pltpu.sync_copy(tab_hbm.at[i_vmem.at[0]], o_vmem)

        pltpu.emit_pipeline(
            body,
            grid=(n // CH,),
            in_specs=[pl.BlockSpec((1, CH), lambda i: (0, i))],
            out_specs=[pl.BlockSpec((CH, 2 * D), lambda i: (i, 0))],
            core_axis_name=("core", "subcore"),
            dimension_semantics=(pltpu.PARALLEL,),
        )(i_hbm, o_hbm)

    return sc_kernel(table, idx2d)


def kernel(target_x, hist_x, hist_card_local_idx, target_card_local_idx,
           card_dense_feats, W1, b1, W2, b2, W3, b3, W4, b4, W5, b5, W6, b6):
    b = target_x.shape[0]
    u = card_dense_feats.shape[0]

    hist_h = _encode(hist_x, W1, b1, W2, b2, blk=2560)

    up = ((u + 127) // 128) * 128
    zsum = jnp.zeros((up, D), jnp.float32)
    zcnt = jnp.zeros((up, 16), jnp.float32)
    psum, pcnt = _sc_segment_sum(
        hist_h, hist_card_local_idx.reshape(1, -1), zsum, zcnt, up)

    dense_p = jnp.pad(card_dense_feats, ((0, up - u), (0, 0)))
    card_h = _card_mlp(psum, pcnt, dense_p,
                       W3[:CF], W3[CF:], b3, W4, b4)

    bp = ((b + CH - 1) // CH) * CH
    tidx = jnp.pad(target_card_local_idx, (0, bp - b)).reshape(1, bp)
    tch = _sc_gather(card_h, tidx)

    w5bp = jnp.concatenate([W5[D:], jnp.zeros((D, D), jnp.float32)], axis=0)
    logits = _head(target_x, tch, W1, b1, W2, b2,
                   W5[:D], w5bp, b5, W6, b6, blk=2000)
    return logits.reshape(b)


# speedup vs baseline: 3.6232x; 1.2083x over previous
import jax
import jax.numpy as jnp
from jax import lax
from jax.experimental import pallas as pl
from jax.experimental.pallas import tpu as pltpu
from jax.experimental.pallas import tpu_sc as plsc

F = 128
D = 64
CF = 5
CH = 128


def _mm(a, b):
    return jnp.dot(a.astype(jnp.bfloat16), b.astype(jnp.bfloat16),
                   preferred_element_type=jnp.float32)


def _enc2_body(xa_ref, xb_ref, w1_ref, b1_ref, w2_ref, b2_ref, o_ref):
    ha = jnp.maximum(_mm(xa_ref[...], w1_ref[...]) + b1_ref[...], 0.0)
    ha = jnp.maximum(_mm(ha, w2_ref[...]) + b2_ref[...], 0.0)
    hb = jnp.maximum(_mm(xb_ref[...], w1_ref[...]) + b1_ref[...], 0.0)
    hb = jnp.maximum(_mm(hb, w2_ref[...]) + b2_ref[...], 0.0)
    o_ref[...] = jnp.concatenate([ha, hb], axis=1)


def _encode_packed(x, w1, b1, w2, b2, blk):
    n = x.shape[0]
    half_blocks = (n // 2) // blk
    return pl.pallas_call(
        _enc2_body,
        grid=(half_blocks,),
        in_specs=[
            pl.BlockSpec((blk, F), lambda i: (i, 0)),
            pl.BlockSpec((blk, F), lambda i: (i + half_blocks, 0)),
            pl.BlockSpec((F, D), lambda i: (0, 0)),
            pl.BlockSpec((1, D), lambda i: (0, 0)),
            pl.BlockSpec((D, D), lambda i: (0, 0)),
            pl.BlockSpec((1, D), lambda i: (0, 0)),
        ],
        out_specs=pl.BlockSpec((blk, 2 * D), lambda i: (i, 0)),
        out_shape=jax.ShapeDtypeStruct((n // 2, 2 * D), jnp.float32),
    )(x, x, w1, b1.reshape(1, D), w2, b2.reshape(1, D))


def _card_body(ps_ref, pc_ref, dense_ref, w3a_ref, w3b_ref, b3_ref,
               w4_ref, b4_ref, o_ref):
    s = ps_ref[0] + ps_ref[1]
    cnt = pc_ref[0, :, 0:1] + pc_ref[1, :, 0:1]
    agg = s / jnp.maximum(cnt, 1.0)
    ch = _mm(agg, w3b_ref[...]) + b3_ref[...]
    for i in range(CF):
        ch = ch + dense_ref[:, i:i + 1] * w3a_ref[i:i + 1, :]
    ch = jnp.maximum(ch, 0.0)
    h = jnp.maximum(_mm(ch, w4_ref[...]) + b4_ref[...], 0.0)
    o_ref[...] = jnp.concatenate([h, jnp.zeros_like(h)], axis=1)


def _card_mlp(psum, pcnt, dense, w3a, w3b, b3, w4, b4):
    u = dense.shape[0]
    return pl.pallas_call(
        _card_body,
        out_shape=jax.ShapeDtypeStruct((u, 2 * D), jnp.float32),
    )(psum, pcnt, dense, w3a, w3b, b3.reshape(1, D), w4, b4.reshape(1, D))


def _head_body(x_ref, tch_ref, w1_ref, b1_ref, w2_ref, b2_ref,
               w5a_ref, w5b_ref, b5_ref, w6_ref, b6_ref, o_ref):
    th = jnp.maximum(_mm(x_ref[...], w1_ref[...]) + b1_ref[...], 0.0)
    th = jnp.maximum(_mm(th, w2_ref[...]) + b2_ref[...], 0.0)
    h = _mm(th, w5a_ref[...]) + _mm(tch_ref[...], w5b_ref[...])
    h = jnp.maximum(h + b5_ref[...], 0.0)
    o_ref[...] = _mm(h, w6_ref[...]) + b6_ref[...]


def _head(target_x, tch, w1, b1, w2, b2, w5a, w5bp, b5, w6, b6, blk):
    n = target_x.shape[0]
    return pl.pallas_call(
        _head_body,
        grid=(n // blk,),
        in_specs=[
            pl.BlockSpec((blk, F), lambda i: (i, 0)),
            pl.BlockSpec((blk, 2 * D), lambda i: (i, 0)),
            pl.BlockSpec((F, D), lambda i: (0, 0)),
            pl.BlockSpec((1, D), lambda i: (0, 0)),
            pl.BlockSpec((D, D), lambda i: (0, 0)),
            pl.BlockSpec((1, D), lambda i: (0, 0)),
            pl.BlockSpec((D, D), lambda i: (0, 0)),
            pl.BlockSpec((2 * D, D), lambda i: (0, 0)),
            pl.BlockSpec((1, D), lambda i: (0, 0)),
            pl.BlockSpec((D, 1), lambda i: (0, 0)),
            pl.BlockSpec((1, 1), lambda i: (0, 0)),
        ],
        out_specs=pl.BlockSpec((blk, 1), lambda i: (i, 0)),
        out_shape=jax.ShapeDtypeStruct((n, 1), jnp.float32),
    )(target_x, tch, w1, b1.reshape(1, D), w2, b2.reshape(1, D),
      w5a, w5bp, b5.reshape(1, D), w6, b6.reshape(1, 1))


_VMESH = plsc.VectorSubcoreMesh(core_axis_name="core", subcore_axis_name="subcore")
_SC_PARAMS = pltpu.CompilerParams(use_tc_tiling_on_sc=False)


def _sc_segment_sum(hh, idx2d, zsum, zcnt, u):
    h = hh.shape[0]
    rows_per_sub = u // 16

    @pl.kernel(
        out_type=(jax.ShapeDtypeStruct((2, u, D), jnp.float32),
                  jax.ShapeDtypeStruct((2, u, 16), jnp.float32)),
        mesh=_VMESH,
        compiler_params=_SC_PARAMS,
        scratch_types=[
            pltpu.VMEM_SHARED((u, D), jnp.float32),
            pltpu.VMEM_SHARED((u, 16), jnp.float32),
            pltpu.VMEM((CH, 16), jnp.float32),
        ],
    )
    def sc_kernel(hh_hbm, idx_hbm, zsum_hbm, zcnt_hbm, osum_hbm, ocnt_hbm,
                  acc_sum, acc_cnt, ones_v):
        cid = lax.axis_index("core")
        sid = lax.axis_index("subcore")

        @pl.loop(0, CH)
        def _(i):
            ones_v.at[pl.ds(i, 1), :][...] = jnp.ones((1, 16), jnp.float32)

        sl = pl.ds(sid * rows_per_sub, rows_per_sub)
        pltpu.sync_copy(zsum_hbm.at[sl], acc_sum.at[sl])
        pltpu.sync_copy(zcnt_hbm.at[sl], acc_cnt.at[sl])
        plsc.subcore_barrier()

        def body(x_vmem, i_vmem):
            pltpu.sync_copy(x_vmem, acc_sum.at[i_vmem.at[0]], add=True)
            pltpu.sync_copy(ones_v, acc_cnt.at[i_vmem.at[0]], add=True)

        pltpu.emit_pipeline(
            body,
            grid=(h // CH,),
            in_specs=[
                pl.BlockSpec((CH, D), lambda i: (i, 0)),
                pl.BlockSpec((1, CH), lambda i: (0, i)),
            ],
            out_specs=[],
            core_axis_name=("core", "subcore"),
            dimension_semantics=(pltpu.PARALLEL,),
        )(hh_hbm, idx_hbm)

        plsc.subcore_barrier()
        pltpu.sync_copy(acc_sum.at[sl], osum_hbm.at[cid, sl])
        pltpu.sync_copy(acc_cnt.at[sl], ocnt_hbm.at[cid, sl])

    return sc_kernel(hh, idx2d, zsum, zcnt)


def _sc_gather(table, idx2d):
    n = idx2d.shape[1]

    @pl.kernel(
        out_type=jax.ShapeDtypeStruct((n, 2 * D), jnp.float32),
        mesh=_VMESH,
    )
    def sc_kernel(tab_hbm, i_hbm, o_hbm):
        def body(i_vmem, o_vmem):
            pltpu.sync_copy(tab_hbm.at[i_vmem.at[0]], o_vmem)

        pltpu.emit_pipeline(
            body,
            grid=(n // CH,),
            in_specs=[pl.BlockSpec((1, CH), lambda i: (0, i))],
            out_specs=[pl.BlockSpec((CH, 2 * D), lambda i: (i, 0))],
            core_axis_name=("core", "subcore"),
            dimension_semantics=(pltpu.PARALLEL,),
        )(i_hbm, o_hbm)

    return sc_kernel(table, idx2d)


def kernel(target_x, hist_x, hist_card_local_idx, target_card_local_idx,
           card_dense_feats, W1, b1, W2, b2, W3, b3, W4, b4, W5, b5, W6, b6):
    b = target_x.shape[0]
    u = card_dense_feats.shape[0]
    h = hist_x.shape[0]

    hist_h2 = _encode_packed(hist_x, W1, b1, W2, b2, blk=3200)
    hist_h = hist_h2.reshape(h, D)
    idx = hist_card_local_idx
    idx_perm = jnp.stack([idx[:h // 2], idx[h // 2:]], axis=1).reshape(1, h)

    up = ((u + 127) // 128) * 128
    zsum = jnp.zeros((up, D), jnp.float32)
    zcnt = jnp.zeros((up, 16), jnp.float32)
    psum, pcnt = _sc_segment_sum(hist_h, idx_perm, zsum, zcnt, up)

    dense_p = jnp.pad(card_dense_feats, ((0, up - u), (0, 0)))
    card_h = _card_mlp(psum, pcnt, dense_p,
                       W3[:CF], W3[CF:], b3, W4, b4)

    bp = ((b + CH - 1) // CH) * CH
    tidx = jnp.pad(target_card_local_idx, (0, bp - b)).reshape(1, bp)
    tch = _sc_gather(card_h, tidx)

    w5bp = jnp.concatenate([W5[D:], jnp.zeros((D, D), jnp.float32)], axis=0)
    logits = _head(target_x, tch, W1, b1, W2, b2,
                   W5[:D], w5bp, b5, W6, b6, blk=5000)
    return logits.reshape(b)


# speedup vs baseline: 4.7801x; 1.3193x over previous
import jax
import jax.numpy as jnp
from jax import lax
from jax.experimental import pallas as pl
from jax.experimental.pallas import tpu as pltpu
from jax.experimental.pallas import tpu_sc as plsc

F = 128
D = 64
CF = 5
CH = 128


def _mm(a, b):
    return jnp.dot(a.astype(jnp.bfloat16), b.astype(jnp.bfloat16),
                   preferred_element_type=jnp.float32)


def _enc2_body(xa_ref, xb_ref, w1_ref, b1_ref, w2_ref, b2_ref, o_ref):
    ha = jnp.maximum(_mm(xa_ref[...], w1_ref[...]) + b1_ref[...], 0.0)
    ha = jnp.maximum(_mm(ha, w2_ref[...]) + b2_ref[...], 0.0)
    hb = jnp.maximum(_mm(xb_ref[...], w1_ref[...]) + b1_ref[...], 0.0)
    hb = jnp.maximum(_mm(hb, w2_ref[...]) + b2_ref[...], 0.0)
    o_ref[...] = jnp.concatenate([ha, hb], axis=1)


def _encode_packed(x, w1, b1, w2, b2, blk):
    n = x.shape[0]
    half_blocks = (n // 2) // blk
    return pl.pallas_call(
        _enc2_body,
        grid=(half_blocks,),
        in_specs=[
            pl.BlockSpec((blk, F), lambda i: (i, 0)),
            pl.BlockSpec((blk, F), lambda i: (i + half_blocks, 0)),
            pl.BlockSpec((F, D), lambda i: (0, 0)),
            pl.BlockSpec((1, D), lambda i: (0, 0)),
            pl.BlockSpec((D, D), lambda i: (0, 0)),
            pl.BlockSpec((1, D), lambda i: (0, 0)),
        ],
        out_specs=pl.BlockSpec((blk, 2 * D), lambda i: (i, 0)),
        out_shape=jax.ShapeDtypeStruct((n // 2, 2 * D), jnp.float32),
    )(x, x, w1, b1.reshape(1, D), w2, b2.reshape(1, D))


def _card_body(ps_ref, pc_ref, dense_ref, w3a_ref, w3b_ref, b3_ref,
               w4_ref, b4_ref, o_ref):
    s = ps_ref[0] + ps_ref[1]
    cnt = pc_ref[0, :, 0:1] + pc_ref[1, :, 0:1]
    agg = s / jnp.maximum(cnt, 1.0)
    ch = _mm(agg, w3b_ref[...]) + b3_ref[...]
    for i in range(CF):
        ch = ch + dense_ref[:, i:i + 1] * w3a_ref[i:i + 1, :]
    ch = jnp.maximum(ch, 0.0)
    h = jnp.maximum(_mm(ch, w4_ref[...]) + b4_ref[...], 0.0)
    o_ref[...] = jnp.concatenate([h, jnp.zeros_like(h)], axis=1)


def _card_mlp(psum, pcnt, dense, w3a, w3b, b3, w4, b4):
    u = dense.shape[0]
    return pl.pallas_call(
        _card_body,
        out_shape=jax.ShapeDtypeStruct((u, 2 * D), jnp.float32),
    )(psum, pcnt, dense, w3a, w3b, b3.reshape(1, D), w4, b4.reshape(1, D))


def _head_body(x_ref, tch_ref, w1_ref, b1_ref, w2_ref, b2_ref,
               w5a_ref, w5b_ref, b5_ref, w6_ref, b6_ref, o_ref):
    th = jnp.maximum(_mm(x_ref[...], w1_ref[...]) + b1_ref[...], 0.0)
    th = jnp.maximum(_mm(th, w2_ref[...]) + b2_ref[...], 0.0)
    h = _mm(th, w5a_ref[...]) + _mm(tch_ref[...], w5b_ref[...])
    h = jnp.maximum(h + b5_ref[...], 0.0)
    o_ref[...] = _mm(h, w6_ref[...]) + b6_ref[...]


def _head(target_x, tch, w1, b1, w2, b2, w5a, w5bp, b5, w6, b6, blk):
    n = target_x.shape[0]
    return pl.pallas_call(
        _head_body,
        grid=(n // blk,),
        in_specs=[
            pl.BlockSpec((blk, F), lambda i: (i, 0)),
            pl.BlockSpec((blk, 2 * D), lambda i: (i, 0)),
            pl.BlockSpec((F, D), lambda i: (0, 0)),
            pl.BlockSpec((1, D), lambda i: (0, 0)),
            pl.BlockSpec((D, D), lambda i: (0, 0)),
            pl.BlockSpec((1, D), lambda i: (0, 0)),
            pl.BlockSpec((D, D), lambda i: (0, 0)),
            pl.BlockSpec((2 * D, D), lambda i: (0, 0)),
            pl.BlockSpec((1, D), lambda i: (0, 0)),
            pl.BlockSpec((D, 1), lambda i: (0, 0)),
            pl.BlockSpec((1, 1), lambda i: (0, 0)),
        ],
        out_specs=pl.BlockSpec((blk, 1), lambda i: (i, 0)),
        out_shape=jax.ShapeDtypeStruct((n, 1), jnp.float32),
    )(target_x, tch, w1, b1.reshape(1, D), w2, b2.reshape(1, D),
      w5a, w5bp, b5.reshape(1, D), w6, b6.reshape(1, 1))


_VMESH = plsc.VectorSubcoreMesh(core_axis_name="core", subcore_axis_name="subcore")
_SC_PARAMS = pltpu.CompilerParams(use_tc_tiling_on_sc=False)


def _sc_segment_sum(hh2, idx2d, zsum, zcnt, u):
    h = hh2.shape[0] * 2
    rows_per_sub = u // 16

    @pl.kernel(
        out_type=(jax.ShapeDtypeStruct((2, u, D), jnp.float32),
                  jax.ShapeDtypeStruct((2, u, 16), jnp.float32)),
        mesh=_VMESH,
        compiler_params=_SC_PARAMS,
        scratch_types=[
            pltpu.VMEM_SHARED((u, D), jnp.float32),
            pltpu.VMEM_SHARED((u, 16), jnp.float32),
            pltpu.VMEM((CH // 2, 16), jnp.float32),
        ],
    )
    def sc_kernel(hh2_hbm, idx_hbm, zsum_hbm, zcnt_hbm, osum_hbm, ocnt_hbm,
                  acc_sum, acc_cnt, ones_v):
        cid = lax.axis_index("core")
        sid = lax.axis_index("subcore")

        @pl.loop(0, CH // 2)
        def _(i):
            ones_v.at[pl.ds(i, 1), :][...] = jnp.ones((1, 16), jnp.float32)

        sl = pl.ds(sid * rows_per_sub, rows_per_sub)
        pltpu.sync_copy(zsum_hbm.at[sl], acc_sum.at[sl])
        pltpu.sync_copy(zcnt_hbm.at[sl], acc_cnt.at[sl])
        plsc.subcore_barrier()

        n_chunks = h // CH

        def body(xl_vmem, xr_vmem, il_vmem, ir_vmem):
            pltpu.sync_copy(xl_vmem, acc_sum.at[il_vmem.at[0]], add=True)
            pltpu.sync_copy(xr_vmem, acc_sum.at[ir_vmem.at[0]], add=True)
            pltpu.sync_copy(ones_v, acc_cnt.at[il_vmem.at[0]], add=True)
            pltpu.sync_copy(ones_v, acc_cnt.at[ir_vmem.at[0]], add=True)

        pltpu.emit_pipeline(
            body,
            grid=(n_chunks,),
            in_specs=[
                pl.BlockSpec((CH // 2, D), lambda i: (i, 0)),
                pl.BlockSpec((CH // 2, D), lambda i: (i, 1)),
                pl.BlockSpec((1, CH // 2), lambda i: (0, i)),
                pl.BlockSpec((1, CH // 2), lambda i, n=n_chunks: (0, i + n)),
            ],
            out_specs=[],
            core_axis_name=("core", "subcore"),
            dimension_semantics=(pltpu.PARALLEL,),
        )(hh2_hbm, hh2_hbm, idx_hbm, idx_hbm)

        plsc.subcore_barrier()
        pltpu.sync_copy(acc_sum.at[sl], osum_hbm.at[cid, sl])
        pltpu.sync_copy(acc_cnt.at[sl], ocnt_hbm.at[cid, sl])

    return sc_kernel(hh2, idx2d, zsum, zcnt)


def _sc_gather(table, idx2d):
    n = idx2d.shape[1]

    @pl.kernel(
        out_type=jax.ShapeDtypeStruct((n, 2 * D), jnp.float32),
        mesh=_VMESH,
    )
    def sc_kernel(tab_hbm, i_hbm, o_hbm):
        def body(i_vmem, o_vmem):
            pltpu.sync_copy(tab_hbm.at[i_vmem.at[0]], o_vmem)

        pltpu.emit_pipeline(
            body,
            grid=(n // CH,),
            in_specs=[pl.BlockSpec((1, CH), lambda i: (0, i))],
            out_specs=[pl.BlockSpec((CH, 2 * D), lambda i: (i, 0))],
            core_axis_name=("core", "subcore"),
            dimension_semantics=(pltpu.PARALLEL,),
        )(i_hbm, o_hbm)

    return sc_kernel(table, idx2d)


def kernel(target_x, hist_x, hist_card_local_idx, target_card_local_idx,
           card_dense_feats, W1, b1, W2, b2, W3, b3, W4, b4, W5, b5, W6, b6):
    b = target_x.shape[0]
    u = card_dense_feats.shape[0]
    h = hist_x.shape[0]

    hist_h2 = _encode_packed(hist_x, W1, b1, W2, b2, blk=3200)

    up = ((u + 127) // 128) * 128
    zsum = jnp.zeros((up, D), jnp.float32)
    zcnt = jnp.zeros((up, 16), jnp.float32)
    psum, pcnt = _sc_segment_sum(
        hist_h2, hist_card_local_idx.reshape(1, h), zsum, zcnt, up)

    dense_p = jnp.pad(card_dense_feats, ((0, up - u), (0, 0)))
    card_h = _card_mlp(psum, pcnt, dense_p,
                       W3[:CF], W3[CF:], b3, W4, b4)

    bp = ((b + CH - 1) // CH) * CH
    tidx = jnp.pad(target_card_local_idx, (0, bp - b)).reshape(1, bp)
    tch = _sc_gather(card_h, tidx)

    w5bp = jnp.concatenate([W5[D:], jnp.zeros((D, D), jnp.float32)], axis=0)
    logits = _head(target_x, tch, W1, b1, W2, b2,
                   W5[:D], w5bp, b5, W6, b6, blk=5000)
    return logits.reshape(b)


# speedup vs baseline: 5.0089x; 1.0478x over previous
import jax
import jax.numpy as jnp
from jax import lax
from jax.experimental import pallas as pl
from jax.experimental.pallas import tpu as pltpu
from jax.experimental.pallas import tpu_sc as plsc

F = 128
D = 64
CF = 5
CH = 128


def _mm(a, b):
    return jnp.dot(a.astype(jnp.bfloat16), b.astype(jnp.bfloat16),
                   preferred_element_type=jnp.float32)


def _enc2_body(xa_ref, xb_ref, w1_ref, b1_ref, w2_ref, b2_ref, o_ref):
    ha = jnp.maximum(_mm(xa_ref[...], w1_ref[...]) + b1_ref[...], 0.0)
    ha = jnp.maximum(_mm(ha, w2_ref[...]) + b2_ref[...], 0.0)
    hb = jnp.maximum(_mm(xb_ref[...], w1_ref[...]) + b1_ref[...], 0.0)
    hb = jnp.maximum(_mm(hb, w2_ref[...]) + b2_ref[...], 0.0)
    o_ref[...] = jnp.concatenate([ha, hb], axis=1)


def _encode_packed(x, w1, b1, w2, b2, blk):
    n = x.shape[0]
    half_blocks = (n // 2) // blk
    return pl.pallas_call(
        _enc2_body,
        grid=(half_blocks,),
        in_specs=[
            pl.BlockSpec((blk, F), lambda i: (i, 0)),
            pl.BlockSpec((blk, F), lambda i: (i + half_blocks, 0)),
            pl.BlockSpec((F, D), lambda i: (0, 0)),
            pl.BlockSpec((1, D), lambda i: (0, 0)),
            pl.BlockSpec((D, D), lambda i: (0, 0)),
            pl.BlockSpec((1, D), lambda i: (0, 0)),
        ],
        out_specs=pl.BlockSpec((blk, 2 * D), lambda i: (i, 0)),
        out_shape=jax.ShapeDtypeStruct((n // 2, 2 * D), jnp.float32),
    )(x, x, w1, b1.reshape(1, D), w2, b2.reshape(1, D))


def _card_body(ps_ref, pc_ref, dense_ref, w3a_ref, w3b_ref, b3_ref,
               w4_ref, b4_ref, o_ref):
    s = ps_ref[0] + ps_ref[1]
    cnt = pc_ref[0, :, 0:1] + pc_ref[1, :, 0:1]
    agg = s / jnp.maximum(cnt, 1.0)
    ch = _mm(agg, w3b_ref[...]) + b3_ref[...]
    for i in range(CF):
        ch = ch + dense_ref[:, i:i + 1] * w3a_ref[i:i + 1, :]
    ch = jnp.maximum(ch, 0.0)
    h = jnp.maximum(_mm(ch, w4_ref[...]) + b4_ref[...], 0.0)
    o_ref[...] = jnp.concatenate([h, jnp.zeros_like(h)], axis=1)


def _card_mlp(psum, pcnt, dense, w3a, w3b, b3, w4, b4):
    u = dense.shape[0]
    return pl.pallas_call(
        _card_body,
        out_shape=jax.ShapeDtypeStruct((u, 2 * D), jnp.float32),
    )(psum, pcnt, dense, w3a, w3b, b3.reshape(1, D), w4, b4.reshape(1, D))


def _head_body(x_ref, tch_ref, w1_ref, b1_ref, w2_ref, b2_ref,
               w5_ref, b5_ref, w6_ref, b6_ref, o_ref):
    th = jnp.maximum(_mm(x_ref[...], w1_ref[...]) + b1_ref[...], 0.0)
    th = jnp.maximum(_mm(th, w2_ref[...]) + b2_ref[...], 0.0)
    hcat = jnp.concatenate([th, tch_ref[...][:, :D]], axis=1)
    h = jnp.maximum(_mm(hcat, w5_ref[...]) + b5_ref[...], 0.0)
    o_ref[...] = (_mm(h, w6_ref[...]) + b6_ref[...]).reshape(o_ref.shape)


def _head(target_x, tch, w1, b1, w2, b2, w5, b5, w6, b6, blk):
    n = target_x.shape[0]
    return pl.pallas_call(
        _head_body,
        grid=(pl.cdiv(n, blk),),
        in_specs=[
            pl.BlockSpec((blk, F), lambda i: (i, 0)),
            pl.BlockSpec((blk, 2 * D), lambda i: (i, 0)),
            pl.BlockSpec((F, D), lambda i: (0, 0)),
            pl.BlockSpec((1, D), lambda i: (0, 0)),
            pl.BlockSpec((D, D), lambda i: (0, 0)),
            pl.BlockSpec((1, D), lambda i: (0, 0)),
            pl.BlockSpec((F, D), lambda i: (0, 0)),
            pl.BlockSpec((1, D), lambda i: (0, 0)),
            pl.BlockSpec((D, 1), lambda i: (0, 0)),
            pl.BlockSpec((1, 1), lambda i: (0, 0)),
        ],
        out_specs=pl.BlockSpec((blk,), lambda i: (i,)),
        out_shape=jax.ShapeDtypeStruct((n,), jnp.float32),
    )(target_x, tch, w1, b1.reshape(1, D), w2, b2.reshape(1, D),
      w5, b5.reshape(1, D), w6, b6.reshape(1, 1))


_VMESH = plsc.VectorSubcoreMesh(core_axis_name="core", subcore_axis_name="subcore")
_SC_PARAMS = pltpu.CompilerParams(use_tc_tiling_on_sc=False)


def _sc_segment_sum(hh2, idx2d, zsum, zcnt, u):
    h = hh2.shape[0] * 2
    rows_per_sub = u // 16

    @pl.kernel(
        out_type=(jax.ShapeDtypeStruct((2, u, D), jnp.float32),
                  jax.ShapeDtypeStruct((2, u, 16), jnp.float32)),
        mesh=_VMESH,
        compiler_params=_SC_PARAMS,
        scratch_types=[
            pltpu.VMEM_SHARED((u, D), jnp.float32),
            pltpu.VMEM_SHARED((u, 16), jnp.float32),
            pltpu.VMEM((CH // 2, 16), jnp.float32),
        ],
    )
    def sc_kernel(hh2_hbm, idx_hbm, zsum_hbm, zcnt_hbm, osum_hbm, ocnt_hbm,
                  acc_sum, acc_cnt, ones_v):
        cid = lax.axis_index("core")
        sid = lax.axis_index("subcore")

        @pl.loop(0, CH // 2)
        def _(i):
            ones_v.at[pl.ds(i, 1), :][...] = jnp.ones((1, 16), jnp.float32)

        sl = pl.ds(sid * rows_per_sub, rows_per_sub)
        pltpu.sync_copy(zsum_hbm.at[sl], acc_sum.at[sl])
        pltpu.sync_copy(zcnt_hbm.at[sl], acc_cnt.at[sl])
        plsc.subcore_barrier()

        n_chunks = h // CH

        def body(xl_vmem, xr_vmem, il_vmem, ir_vmem):
            pltpu.sync_copy(xl_vmem, acc_sum.at[il_vmem.at[0]], add=True)
            pltpu.sync_copy(xr_vmem, acc_sum.at[ir_vmem.at[0]], add=True)
            pltpu.sync_copy(ones_v, acc_cnt.at[il_vmem.at[0]], add=True)
            pltpu.sync_copy(ones_v, acc_cnt.at[ir_vmem.at[0]], add=True)

        pltpu.emit_pipeline(
            body,
            grid=(n_chunks,),
            in_specs=[
                pl.BlockSpec((CH // 2, D), lambda i: (i, 0)),
                pl.BlockSpec((CH // 2, D), lambda i: (i, 1)),
                pl.BlockSpec((1, CH // 2), lambda i: (0, i)),
                pl.BlockSpec((1, CH // 2), lambda i, n=n_chunks: (0, i + n)),
            ],
            out_specs=[],
            core_axis_name=("core", "subcore"),
            dimension_semantics=(pltpu.PARALLEL,),
        )(hh2_hbm, hh2_hbm, idx_hbm, idx_hbm)

        plsc.subcore_barrier()
        pltpu.sync_copy(acc_sum.at[sl], osum_hbm.at[cid, sl])
        pltpu.sync_copy(acc_cnt.at[sl], ocnt_hbm.at[cid, sl])

    return sc_kernel(hh2, idx2d, zsum, zcnt)


def _sc_gather(table, idx2d):
    n = idx2d.shape[1]

    @pl.kernel(
        out_type=jax.ShapeDtypeStruct((n, 2 * D), jnp.float32),
        mesh=_VMESH,
    )
    def sc_kernel(tab_hbm, i_hbm, o_hbm):
        def body(i_vmem, o_vmem):
            pltpu.sync_copy(tab_hbm.at[i_vmem.at[0]], o_vmem)

        pltpu.emit_pipeline(
            body,
            grid=(n // CH,),
            in_specs=[pl.BlockSpec((1, CH), lambda i: (0, i))],
            out_specs=[pl.BlockSpec((CH, 2 * D), lambda i: (i, 0))],
            core_axis_name=("core", "subcore"),
            dimension_semantics=(pltpu.PARALLEL,),
        )(i_hbm, o_hbm)

    return sc_kernel(table, idx2d)


def kernel(target_x, hist_x, hist_card_local_idx, target_card_local_idx,
           card_dense_feats, W1, b1, W2, b2, W3, b3, W4, b4, W5, b5, W6, b6):
    b = target_x.shape[0]
    u = card_dense_feats.shape[0]
    h = hist_x.shape[0]

    hist_h2 = _encode_packed(hist_x, W1, b1, W2, b2, blk=3200)

    up = ((u + 127) // 128) * 128
    zsum = jnp.zeros((up, D), jnp.float32)
    zcnt = jnp.zeros((up, 16), jnp.float32)
    psum, pcnt = _sc_segment_sum(
        hist_h2, hist_card_local_idx.reshape(1, h), zsum, zcnt, up)

    dense_p = jnp.pad(card_dense_feats, ((0, up - u), (0, 0)))
    card_h = _card_mlp(psum, pcnt, dense_p,
                       W3[:CF], W3[CF:], b3, W4, b4)

    bp = ((b + CH - 1) // CH) * CH
    tidx = jnp.pad(target_card_local_idx, (0, bp - b)).reshape(1, bp)
    tch = _sc_gather(card_h, tidx)

    return _head(target_x, tch, W1, b1, W2, b2, W5, b5, W6, b6, blk=2048)


# speedup vs baseline: 5.2228x; 1.0427x over previous
import jax
import jax.numpy as jnp
from jax import lax
from jax.experimental import pallas as pl
from jax.experimental.pallas import tpu as pltpu
from jax.experimental.pallas import tpu_sc as plsc

F = 128
D = 64
CF = 5
CH = 128


def _mm(a, b):
    return jnp.dot(a.astype(jnp.bfloat16), b.astype(jnp.bfloat16),
                   preferred_element_type=jnp.float32)


def _enc2_body(xa_ref, xb_ref, w1_ref, b1_ref, w2_ref, b2_ref, o_ref):
    ha = jnp.maximum(_mm(xa_ref[...], w1_ref[...]) + b1_ref[...], 0.0)
    ha = jnp.maximum(_mm(ha, w2_ref[...]) + b2_ref[...], 0.0)
    hb = jnp.maximum(_mm(xb_ref[...], w1_ref[...]) + b1_ref[...], 0.0)
    hb = jnp.maximum(_mm(hb, w2_ref[...]) + b2_ref[...], 0.0)
    o_ref[...] = jnp.concatenate([ha, hb], axis=1)


def _encode_packed(x, w1, b1, w2, b2, blk, n_rows, row_off):
    half_blocks = (n_rows // 2) // blk
    off_a = row_off // blk
    off_b = (row_off + n_rows // 2) // blk
    return pl.pallas_call(
        _enc2_body,
        grid=(half_blocks,),
        in_specs=[
            pl.BlockSpec((blk, F), lambda i: (i + off_a, 0)),
            pl.BlockSpec((blk, F), lambda i: (i + off_b, 0)),
            pl.BlockSpec((F, D), lambda i: (0, 0)),
            pl.BlockSpec((1, D), lambda i: (0, 0)),
            pl.BlockSpec((D, D), lambda i: (0, 0)),
            pl.BlockSpec((1, D), lambda i: (0, 0)),
        ],
        out_specs=pl.BlockSpec((blk, 2 * D), lambda i: (i, 0)),
        out_shape=jax.ShapeDtypeStruct((n_rows // 2, 2 * D), jnp.float32),
    )(x, x, w1, b1.reshape(1, D), w2, b2.reshape(1, D))


def _card_body(psa_ref, pca_ref, psb_ref, pcb_ref, dense_ref,
               w3a_ref, w3b_ref, b3_ref, w4_ref, b4_ref, o_ref):
    s = psa_ref[0] + psa_ref[1] + psb_ref[0] + psb_ref[1]
    cnt = (pca_ref[0, :, 0:1] + pca_ref[1, :, 0:1]
           + pcb_ref[0, :, 0:1] + pcb_ref[1, :, 0:1])
    agg = s / jnp.maximum(cnt, 1.0)
    ch = _mm(agg, w3b_ref[...]) + b3_ref[...]
    for i in range(CF):
        ch = ch + dense_ref[:, i:i + 1] * w3a_ref[i:i + 1, :]
    ch = jnp.maximum(ch, 0.0)
    h = jnp.maximum(_mm(ch, w4_ref[...]) + b4_ref[...], 0.0)
    o_ref[...] = jnp.concatenate([h, jnp.zeros_like(h)], axis=1)


def _card_mlp(psum_a, pcnt_a, psum_b, pcnt_b, dense, w3a, w3b, b3, w4, b4):
    u = dense.shape[0]
    blk = u // 4
    return pl.pallas_call(
        _card_body,
        grid=(4,),
        in_specs=[
            pl.BlockSpec((2, blk, D), lambda i: (0, i, 0)),
            pl.BlockSpec((2, blk, 16), lambda i: (0, i, 0)),
            pl.BlockSpec((2, blk, D), lambda i: (0, i, 0)),
            pl.BlockSpec((2, blk, 16), lambda i: (0, i, 0)),
            pl.BlockSpec((blk, CF), lambda i: (i, 0)),
            pl.BlockSpec((CF, D), lambda i: (0, 0)),
            pl.BlockSpec((D, D), lambda i: (0, 0)),
            pl.BlockSpec((1, D), lambda i: (0, 0)),
            pl.BlockSpec((D, D), lambda i: (0, 0)),
            pl.BlockSpec((1, D), lambda i: (0, 0)),
        ],
        out_specs=pl.BlockSpec((blk, 2 * D), lambda i: (i, 0)),
        out_shape=jax.ShapeDtypeStruct((u, 2 * D), jnp.float32),
    )(psum_a, pcnt_a, psum_b, pcnt_b, dense,
      w3a, w3b, b3.reshape(1, D), w4, b4.reshape(1, D))


def _head_body(x_ref, tch_ref, w1_ref, b1_ref, w2_ref, b2_ref,
               w5_ref, b5_ref, w6_ref, b6_ref, o_ref):
    th = jnp.maximum(_mm(x_ref[...], w1_ref[...]) + b1_ref[...], 0.0)
    th = jnp.maximum(_mm(th, w2_ref[...]) + b2_ref[...], 0.0)
    hcat = jnp.concatenate([th, tch_ref[...][:, :D]], axis=1)
    h = jnp.maximum(_mm(hcat, w5_ref[...]) + b5_ref[...], 0.0)
    o_ref[...] = (_mm(h, w6_ref[...]) + b6_ref[...]).reshape(o_ref.shape)


def _head(target_x, tch, w1, b1, w2, b2, w5, b5, w6, b6, blk):
    n = target_x.shape[0]
    return pl.pallas_call(
        _head_body,
        grid=(pl.cdiv(n, blk),),
        in_specs=[
            pl.BlockSpec((blk, F), lambda i: (i, 0)),
            pl.BlockSpec((blk, 2 * D), lambda i: (i, 0)),
            pl.BlockSpec((F, D), lambda i: (0, 0)),
            pl.BlockSpec((1, D), lambda i: (0, 0)),
            pl.BlockSpec((D, D), lambda i: (0, 0)),
            pl.BlockSpec((1, D), lambda i: (0, 0)),
            pl.BlockSpec((F, D), lambda i: (0, 0)),
            pl.BlockSpec((1, D), lambda i: (0, 0)),
            pl.BlockSpec((D, 1), lambda i: (0, 0)),
            pl.BlockSpec((1, 1), lambda i: (0, 0)),
        ],
        out_specs=pl.BlockSpec((blk,), lambda i: (i,)),
        out_shape=jax.ShapeDtypeStruct((n,), jnp.float32),
    )(target_x, tch, w1, b1.reshape(1, D), w2, b2.reshape(1, D),
      w5, b5.reshape(1, D), w6, b6.reshape(1, 1))


_VMESH = plsc.VectorSubcoreMesh(core_axis_name="core", subcore_axis_name="subcore")
_SC_PARAMS = pltpu.CompilerParams(use_tc_tiling_on_sc=False)


def _sc_segment_sum(hh2, idx2d, zsum, zcnt, u):
    h = hh2.shape[0] * 2
    rows_per_sub = u // 16

    @pl.kernel(
        out_type=(jax.ShapeDtypeStruct((2, u, D), jnp.float32),
                  jax.ShapeDtypeStruct((2, u, 16), jnp.float32)),
        mesh=_VMESH,
        compiler_params=_SC_PARAMS,
        scratch_types=[
            pltpu.VMEM_SHARED((u, D), jnp.float32),
            pltpu.VMEM_SHARED((u, 16), jnp.float32),
            pltpu.VMEM((CH // 2, 16), jnp.float32),
        ],
    )
    def sc_kernel(hh2_hbm, idx_hbm, zsum_hbm, zcnt_hbm, osum_hbm, ocnt_hbm,
                  acc_sum, acc_cnt, ones_v):
        cid = lax.axis_index("core")
        sid = lax.axis_index("subcore")

        @pl.loop(0, CH // 2)
        def _(i):
            ones_v.at[pl.ds(i, 1), :][...] = jnp.ones((1, 16), jnp.float32)

        sl = pl.ds(sid * rows_per_sub, rows_per_sub)
        pltpu.sync_copy(zsum_hbm.at[sl], acc_sum.at[sl])
        pltpu.sync_copy(zcnt_hbm.at[sl], acc_cnt.at[sl])
        plsc.subcore_barrier()

        n_chunks = h // CH

        def body(xl_vmem, xr_vmem, il_vmem, ir_vmem):
            pltpu.sync_copy(xl_vmem, acc_sum.at[il_vmem.at[0]], add=True)
            pltpu.sync_copy(xr_vmem, acc_sum.at[ir_vmem.at[0]], add=True)
            pltpu.sync_copy(ones_v, acc_cnt.at[il_vmem.at[0]], add=True)
            pltpu.sync_copy(ones_v, acc_cnt.at[ir_vmem.at[0]], add=True)

        pltpu.emit_pipeline(
            body,
            grid=(n_chunks,),
            in_specs=[
                pl.BlockSpec((CH // 2, D), lambda i: (i, 0)),
                pl.BlockSpec((CH // 2, D), lambda i: (i, 1)),
                pl.BlockSpec((1, CH // 2), lambda i: (0, i)),
                pl.BlockSpec((1, CH // 2), lambda i, n=n_chunks: (0, i + n)),
            ],
            out_specs=[],
            core_axis_name=("core", "subcore"),
            dimension_semantics=(pltpu.PARALLEL,),
        )(hh2_hbm, hh2_hbm, idx_hbm, idx_hbm)

        plsc.subcore_barrier()
        pltpu.sync_copy(acc_sum.at[sl], osum_hbm.at[cid, sl])
        pltpu.sync_copy(acc_cnt.at[sl], ocnt_hbm.at[cid, sl])

    return sc_kernel(hh2, idx2d, zsum, zcnt)


def _sc_gather(table, idx2d):
    n = idx2d.shape[1]

    @pl.kernel(
        out_type=jax.ShapeDtypeStruct((n, 2 * D), jnp.float32),
        mesh=_VMESH,
    )
    def sc_kernel(tab_hbm, i_hbm, o_hbm):
        def body(i_vmem, o_vmem):
            pltpu.sync_copy(tab_hbm.at[i_vmem.at[0]], o_vmem)

        pltpu.emit_pipeline(
            body,
            grid=(n // CH,),
            in_specs=[pl.BlockSpec((1, CH), lambda i: (0, i))],
            out_specs=[pl.BlockSpec((CH, 2 * D), lambda i: (i, 0))],
            core_axis_name=("core", "subcore"),
            dimension_semantics=(pltpu.PARALLEL,),
        )(i_hbm, o_hbm)

    return sc_kernel(table, idx2d)


def kernel(target_x, hist_x, hist_card_local_idx, target_card_local_idx,
           card_dense_feats, W1, b1, W2, b2, W3, b3, W4, b4, W5, b5, W6, b6):
    b = target_x.shape[0]
    u = card_dense_feats.shape[0]
    h = hist_x.shape[0]

    h2 = h // 2
    idx = hist_card_local_idx
    hh2_a = _encode_packed(hist_x, W1, b1, W2, b2, blk=3200,
                           n_rows=h2, row_off=0)
    hh2_b = _encode_packed(hist_x, W1, b1, W2, b2, blk=3200,
                           n_rows=h2, row_off=h2)

    up = ((u + 127) // 128) * 128
    zsum = jnp.zeros((up, D), jnp.float32)
    zcnt = jnp.zeros((up, 16), jnp.float32)
    psum_a, pcnt_a = _sc_segment_sum(
        hh2_a, idx[:h2].reshape(1, h2), zsum, zcnt, up)
    psum_b, pcnt_b = _sc_segment_sum(
        hh2_b, idx[h2:].reshape(1, h2), zsum, zcnt, up)

    dense_p = jnp.pad(card_dense_feats, ((0, up - u), (0, 0)))
    card_h = _card_mlp(psum_a, pcnt_a, psum_b, pcnt_b, dense_p,
                       W3[:CF], W3[CF:], b3, W4, b4)

    bp = ((b + CH - 1) // CH) * CH
    tidx = jnp.pad(target_card_local_idx, (0, bp - b)).reshape(1, bp)
    tch = _sc_gather(card_h, tidx)

    return _head(target_x, tch, W1, b1, W2, b2, W5, b5, W6, b6, blk=2048)


# speedup vs baseline: 5.4952x; 1.0522x over previous
import jax
import jax.numpy as jnp
from jax import lax
from jax.experimental import pallas as pl
from jax.experimental.pallas import tpu as pltpu
from jax.experimental.pallas import tpu_sc as plsc

F = 128
D = 64
CF = 5
CH = 128


def _mm(a, b):
    return jnp.dot(a.astype(jnp.bfloat16), b.astype(jnp.bfloat16),
                   preferred_element_type=jnp.float32)


def _enc2_body(xa_ref, xb_ref, w1_ref, b1_ref, w2_ref, b2_ref, o_ref):
    ha = jnp.maximum(_mm(xa_ref[...], w1_ref[...]) + b1_ref[...], 0.0)
    ha = jnp.maximum(_mm(ha, w2_ref[...]) + b2_ref[...], 0.0)
    hb = jnp.maximum(_mm(xb_ref[...], w1_ref[...]) + b1_ref[...], 0.0)
    hb = jnp.maximum(_mm(hb, w2_ref[...]) + b2_ref[...], 0.0)
    o_ref[...] = jnp.concatenate([ha, hb], axis=1)


def _encode_packed(x, w1, b1, w2, b2, blk, n_rows, row_off):
    half_blocks = (n_rows // 2) // blk
    off_a = row_off // blk
    off_b = (row_off + n_rows // 2) // blk
    return pl.pallas_call(
        _enc2_body,
        grid=(half_blocks,),
        in_specs=[
            pl.BlockSpec((blk, F), lambda i: (i + off_a, 0)),
            pl.BlockSpec((blk, F), lambda i: (i + off_b, 0)),
            pl.BlockSpec((F, D), lambda i: (0, 0)),
            pl.BlockSpec((1, D), lambda i: (0, 0)),
            pl.BlockSpec((D, D), lambda i: (0, 0)),
            pl.BlockSpec((1, D), lambda i: (0, 0)),
        ],
        out_specs=pl.BlockSpec((blk, 2 * D), lambda i: (i, 0)),
        out_shape=jax.ShapeDtypeStruct((n_rows // 2, 2 * D), jnp.float32),
    )(x, x, w1, b1.reshape(1, D), w2, b2.reshape(1, D))


def _card_body(psa_ref, pca_ref, psb_ref, pcb_ref, dense_ref,
               w3a_ref, w3b_ref, b3_ref, w4_ref, b4_ref, o_ref):
    s = psa_ref[0] + psa_ref[1] + psb_ref[0] + psb_ref[1]
    cnt = (pca_ref[0, :, 0:1] + pca_ref[1, :, 0:1]
           + pcb_ref[0, :, 0:1] + pcb_ref[1, :, 0:1])
    agg = s / jnp.maximum(cnt, 1.0)
    ch = _mm(agg, w3b_ref[...]) + b3_ref[...]
    for i in range(CF):
        ch = ch + dense_ref[:, i:i + 1] * w3a_ref[i:i + 1, :]
    ch = jnp.maximum(ch, 0.0)
    h = jnp.maximum(_mm(ch, w4_ref[...]) + b4_ref[...], 0.0)
    o_ref[...] = jnp.concatenate([h, jnp.zeros_like(h)], axis=1)


def _card_mlp(psum_a, pcnt_a, psum_b, pcnt_b, dense, w3a, w3b, b3, w4, b4):
    u = dense.shape[0]
    blk = u // 4
    return pl.pallas_call(
        _card_body,
        grid=(4,),
        in_specs=[
            pl.BlockSpec((2, blk, D), lambda i: (0, i, 0)),
            pl.BlockSpec((2, blk, 16), lambda i: (0, i, 0)),
            pl.BlockSpec((2, blk, D), lambda i: (0, i, 0)),
            pl.BlockSpec((2, blk, 16), lambda i: (0, i, 0)),
            pl.BlockSpec((blk, CF), lambda i: (i, 0)),
            pl.BlockSpec((CF, D), lambda i: (0, 0)),
            pl.BlockSpec((D, D), lambda i: (0, 0)),
            pl.BlockSpec((1, D), lambda i: (0, 0)),
            pl.BlockSpec((D, D), lambda i: (0, 0)),
            pl.BlockSpec((1, D), lambda i: (0, 0)),
        ],
        out_specs=pl.BlockSpec((blk, 2 * D), lambda i: (i, 0)),
        out_shape=jax.ShapeDtypeStruct((u, 2 * D), jnp.float32),
    )(psum_a, pcnt_a, psum_b, pcnt_b, dense,
      w3a, w3b, b3.reshape(1, D), w4, b4.reshape(1, D))


def _head_body(x_ref, tch_ref, w1_ref, b1_ref, w2_ref, b2_ref,
               w5_ref, b5_ref, w6_ref, b6_ref, o_ref):
    th = jnp.maximum(_mm(x_ref[...], w1_ref[...]) + b1_ref[...], 0.0)
    th = jnp.maximum(_mm(th, w2_ref[...]) + b2_ref[...], 0.0)
    hcat = jnp.concatenate([th, tch_ref[...][:, :D]], axis=1)
    h = jnp.maximum(_mm(hcat, w5_ref[...]) + b5_ref[...], 0.0)
    o_ref[...] = (_mm(h, w6_ref[...]) + b6_ref[...]).reshape(o_ref.shape)


def _head(target_x, tch, w1, b1, w2, b2, w5, b5, w6, b6, blk):
    n = target_x.shape[0]
    return pl.pallas_call(
        _head_body,
        grid=(pl.cdiv(n, blk),),
        in_specs=[
            pl.BlockSpec((blk, F), lambda i: (i, 0)),
            pl.BlockSpec((blk, 2 * D), lambda i: (i, 0)),
            pl.BlockSpec((F, D), lambda i: (0, 0)),
            pl.BlockSpec((1, D), lambda i: (0, 0)),
            pl.BlockSpec((D, D), lambda i: (0, 0)),
            pl.BlockSpec((1, D), lambda i: (0, 0)),
            pl.BlockSpec((F, D), lambda i: (0, 0)),
            pl.BlockSpec((1, D), lambda i: (0, 0)),
            pl.BlockSpec((D, 1), lambda i: (0, 0)),
            pl.BlockSpec((1, 1), lambda i: (0, 0)),
        ],
        out_specs=pl.BlockSpec((blk,), lambda i: (i,)),
        out_shape=jax.ShapeDtypeStruct((n,), jnp.float32),
    )(target_x, tch, w1, b1.reshape(1, D), w2, b2.reshape(1, D),
      w5, b5.reshape(1, D), w6, b6.reshape(1, 1))


_VMESH = plsc.VectorSubcoreMesh(core_axis_name="core", subcore_axis_name="subcore")
_SC_PARAMS = pltpu.CompilerParams(use_tc_tiling_on_sc=False)


def _sc_segment_sum(hh2, idx2d, u):
    h = hh2.shape[0] * 2
    rows_per_sub = u // 16

    @pl.kernel(
        out_type=(jax.ShapeDtypeStruct((2, u, D), jnp.float32),
                  jax.ShapeDtypeStruct((2, u, 16), jnp.float32)),
        mesh=_VMESH,
        compiler_params=_SC_PARAMS,
        scratch_types=[
            pltpu.VMEM_SHARED((u, D), jnp.float32),
            pltpu.VMEM_SHARED((u, 16), jnp.float32),
            pltpu.VMEM((CH // 2, 16), jnp.float32),
            pltpu.VMEM((rows_per_sub, D), jnp.float32),
            pltpu.VMEM((rows_per_sub, 16), jnp.float32),
        ],
    )
    def sc_kernel(hh2_hbm, idx_hbm, osum_hbm, ocnt_hbm,
                  acc_sum, acc_cnt, ones_v, zero_v, zcnt_v):
        cid = lax.axis_index("core")
        sid = lax.axis_index("subcore")

        @pl.loop(0, CH // 2)
        def _(i):
            ones_v.at[pl.ds(i, 1), :][...] = jnp.ones((1, 16), jnp.float32)

        @pl.loop(0, rows_per_sub)
        def _(i):
            for j in range(D // 16):
                zero_v.at[pl.ds(i, 1), pl.ds(j * 16, 16)][...] = (
                    jnp.zeros((1, 16), jnp.float32))
            zcnt_v.at[pl.ds(i, 1), :][...] = jnp.zeros((1, 16), jnp.float32)

        sl = pl.ds(sid * rows_per_sub, rows_per_sub)
        pltpu.sync_copy(zero_v, acc_sum.at[sl])
        pltpu.sync_copy(zcnt_v, acc_cnt.at[sl])
        plsc.subcore_barrier()

        n_chunks = h // CH

        def body(xl_vmem, xr_vmem, il_vmem, ir_vmem):
            pltpu.sync_copy(xl_vmem, acc_sum.at[il_vmem.at[0]], add=True)
            pltpu.sync_copy(xr_vmem, acc_sum.at[ir_vmem.at[0]], add=True)
            pltpu.sync_copy(ones_v, acc_cnt.at[il_vmem.at[0]], add=True)
            pltpu.sync_copy(ones_v, acc_cnt.at[ir_vmem.at[0]], add=True)

        pltpu.emit_pipeline(
            body,
            grid=(n_chunks,),
            in_specs=[
                pl.BlockSpec((CH // 2, D), lambda i: (i, 0)),
                pl.BlockSpec((CH // 2, D), lambda i: (i, 1)),
                pl.BlockSpec((1, CH // 2), lambda i: (0, i)),
                pl.BlockSpec((1, CH // 2), lambda i, n=n_chunks: (0, i + n)),
            ],
            out_specs=[],
            core_axis_name=("core", "subcore"),
            dimension_semantics=(pltpu.PARALLEL,),
        )(hh2_hbm, hh2_hbm, idx_hbm, idx_hbm)

        plsc.subcore_barrier()
        pltpu.sync_copy(acc_sum.at[sl], osum_hbm.at[cid, sl])
        pltpu.sync_copy(acc_cnt.at[sl], ocnt_hbm.at[cid, sl])

    return sc_kernel(hh2, idx2d)


def _sc_gather(table, idx2d):
    n = idx2d.shape[1]

    @pl.kernel(
        out_type=jax.ShapeDtypeStruct((n, 2 * D), jnp.float32),
        mesh=_VMESH,
    )
    def sc_kernel(tab_hbm, i_hbm, o_hbm):
        def body(i_vmem, o_vmem):
            pltpu.sync_copy(tab_hbm.at[i_vmem.at[0]], o_vmem)

        pltpu.emit_pipeline(
            body,
            grid=(n // CH,),
            in_specs=[pl.BlockSpec((1, CH), lambda i: (0, i))],
            out_specs=[pl.BlockSpec((CH, 2 * D), lambda i: (i, 0))],
            core_axis_name=("core", "subcore"),
            dimension_semantics=(pltpu.PARALLEL,),
        )(i_hbm, o_hbm)

    return sc_kernel(table, idx2d)


def kernel(target_x, hist_x, hist_card_local_idx, target_card_local_idx,
           card_dense_feats, W1, b1, W2, b2, W3, b3, W4, b4, W5, b5, W6, b6):
    b = target_x.shape[0]
    u = card_dense_feats.shape[0]
    h = hist_x.shape[0]

    h2 = h // 2
    idx = hist_card_local_idx
    hh2_a = _encode_packed(hist_x, W1, b1, W2, b2, blk=8000,
                           n_rows=h2, row_off=0)
    hh2_b = _encode_packed(hist_x, W1, b1, W2, b2, blk=8000,
                           n_rows=h2, row_off=h2)

    up = ((u + 127) // 128) * 128
    psum_a, pcnt_a = _sc_segment_sum(hh2_a, idx[:h2].reshape(1, h2), up)
    psum_b, pcnt_b = _sc_segment_sum(hh2_b, idx[h2:].reshape(1, h2), up)

    dense_p = jnp.pad(card_dense_feats, ((0, up - u), (0, 0)))
    card_h = _card_mlp(psum_a, pcnt_a, psum_b, pcnt_b, dense_p,
                       W3[:CF], W3[CF:], b3, W4, b4)

    bp = ((b + CH - 1) // CH) * CH
    tidx = jnp.pad(target_card_local_idx, (0, bp - b)).reshape(1, bp)
    tch = _sc_gather(card_h, tidx)

    return _head(target_x, tch, W1, b1, W2, b2, W5, b5, W6, b6, blk=4096)


# speedup vs baseline: 5.5986x; 1.0188x over previous
import jax
import jax.numpy as jnp
from jax import lax
from jax.experimental import pallas as pl
from jax.experimental.pallas import tpu as pltpu
from jax.experimental.pallas import tpu_sc as plsc

F = 128
D = 64
CF = 5
CH = 128


def _mm(a, b):
    return jnp.dot(a.astype(jnp.bfloat16), b.astype(jnp.bfloat16),
                   preferred_element_type=jnp.float32)


def _enc2_body(xa_ref, xb_ref, w1_ref, b1_ref, w2_ref, b2_ref, o_ref):
    ha = jnp.maximum(_mm(xa_ref[...], w1_ref[...]) + b1_ref[...], 0.0)
    ha = jnp.maximum(_mm(ha, w2_ref[...]) + b2_ref[...], 0.0)
    hb = jnp.maximum(_mm(xb_ref[...], w1_ref[...]) + b1_ref[...], 0.0)
    hb = jnp.maximum(_mm(hb, w2_ref[...]) + b2_ref[...], 0.0)
    o_ref[...] = jnp.concatenate([ha, hb], axis=1)


def _encode_packed(x, w1, b1, w2, b2, blk, n_rows, row_off):
    half_blocks = (n_rows // 2) // blk
    off_a = row_off // blk
    off_b = (row_off + n_rows // 2) // blk
    return pl.pallas_call(
        _enc2_body,
        grid=(half_blocks,),
        in_specs=[
            pl.BlockSpec((blk, F), lambda i: (i + off_a, 0)),
            pl.BlockSpec((blk, F), lambda i: (i + off_b, 0)),
            pl.BlockSpec((F, D), lambda i: (0, 0)),
            pl.BlockSpec((1, D), lambda i: (0, 0)),
            pl.BlockSpec((D, D), lambda i: (0, 0)),
            pl.BlockSpec((1, D), lambda i: (0, 0)),
        ],
        out_specs=pl.BlockSpec((blk, 2 * D), lambda i: (i, 0)),
        out_shape=jax.ShapeDtypeStruct((n_rows // 2, 2 * D), jnp.float32),
    )(x, x, w1, b1.reshape(1, D), w2, b2.reshape(1, D))


def _card_body(psa_ref, pca_ref, psb_ref, pcb_ref, dense_ref,
               w3a_ref, w3b_ref, b3_ref, w4_ref, b4_ref, o_ref):
    lo = pl.program_id(0) < 4
    s128 = psa_ref[0] + psa_ref[1] + psb_ref[0] + psb_ref[1]
    s = jnp.where(lo, s128[:, :D], s128[:, D:])
    cnt = (pca_ref[0, :, 0:1] + pca_ref[1, :, 0:1]
           + pcb_ref[0, :, 0:1] + pcb_ref[1, :, 0:1])
    agg = s / jnp.maximum(cnt, 1.0)
    ch = _mm(agg, w3b_ref[...]) + b3_ref[...]
    for i in range(CF):
        ch = ch + dense_ref[:, i:i + 1] * w3a_ref[i:i + 1, :]
    ch = jnp.maximum(ch, 0.0)
    h = jnp.maximum(_mm(ch, w4_ref[...]) + b4_ref[...], 0.0)
    o_ref[...] = jnp.concatenate([h, jnp.zeros_like(h)], axis=1)


def _card_mlp(psum_a, pcnt_a, psum_b, pcnt_b, dense, w3a, w3b, b3, w4, b4):
    u = dense.shape[0]
    blk = u // 8
    return pl.pallas_call(
        _card_body,
        grid=(8,),
        in_specs=[
            pl.BlockSpec((2, blk, 2 * D), lambda i: (0, jnp.where(i < 4, i, i - 4), 0)),
            pl.BlockSpec((2, blk, 16), lambda i: (0, i, 0)),
            pl.BlockSpec((2, blk, 2 * D), lambda i: (0, jnp.where(i < 4, i, i - 4), 0)),
            pl.BlockSpec((2, blk, 16), lambda i: (0, i, 0)),
            pl.BlockSpec((blk, CF), lambda i: (i, 0)),
            pl.BlockSpec((CF, D), lambda i: (0, 0)),
            pl.BlockSpec((D, D), lambda i: (0, 0)),
            pl.BlockSpec((1, D), lambda i: (0, 0)),
            pl.BlockSpec((D, D), lambda i: (0, 0)),
            pl.BlockSpec((1, D), lambda i: (0, 0)),
        ],
        out_specs=pl.BlockSpec((blk, 2 * D), lambda i: (i, 0)),
        out_shape=jax.ShapeDtypeStruct((u, 2 * D), jnp.float32),
    )(psum_a, pcnt_a, psum_b, pcnt_b, dense,
      w3a, w3b, b3.reshape(1, D), w4, b4.reshape(1, D))


def _head_body(x_ref, tch_ref, w1_ref, b1_ref, w2_ref, b2_ref,
               w5_ref, b5_ref, w6_ref, b6_ref, o_ref):
    th = jnp.maximum(_mm(x_ref[...], w1_ref[...]) + b1_ref[...], 0.0)
    th = jnp.maximum(_mm(th, w2_ref[...]) + b2_ref[...], 0.0)
    hcat = jnp.concatenate([th, tch_ref[...][:, :D]], axis=1)
    h = jnp.maximum(_mm(hcat, w5_ref[...]) + b5_ref[...], 0.0)
    o_ref[...] = (_mm(h, w6_ref[...]) + b6_ref[...]).reshape(o_ref.shape)


def _head(target_x, tch, w1, b1, w2, b2, w5, b5, w6, b6, blk):
    n = target_x.shape[0]
    return pl.pallas_call(
        _head_body,
        grid=(pl.cdiv(n, blk),),
        in_specs=[
            pl.BlockSpec((blk, F), lambda i: (i, 0)),
            pl.BlockSpec((blk, 2 * D), lambda i: (i, 0)),
            pl.BlockSpec((F, D), lambda i: (0, 0)),
            pl.BlockSpec((1, D), lambda i: (0, 0)),
            pl.BlockSpec((D, D), lambda i: (0, 0)),
            pl.BlockSpec((1, D), lambda i: (0, 0)),
            pl.BlockSpec((F, D), lambda i: (0, 0)),
            pl.BlockSpec((1, D), lambda i: (0, 0)),
            pl.BlockSpec((D, 1), lambda i: (0, 0)),
            pl.BlockSpec((1, 1), lambda i: (0, 0)),
        ],
        out_specs=pl.BlockSpec((blk,), lambda i: (i,)),
        out_shape=jax.ShapeDtypeStruct((n,), jnp.float32),
    )(target_x, tch, w1, b1.reshape(1, D), w2, b2.reshape(1, D),
      w5, b5.reshape(1, D), w6, b6.reshape(1, 1))


_VMESH = plsc.VectorSubcoreMesh(core_axis_name="core", subcore_axis_name="subcore")
_SC_PARAMS = pltpu.CompilerParams(use_tc_tiling_on_sc=False)


def _sc_segment_sum(hh2, idx2d, u):
    h = hh2.shape[0] * 2
    rows_per_sub = u // 16

    @pl.kernel(
        out_type=(jax.ShapeDtypeStruct((2, u // 2, 2 * D), jnp.float32),
                  jax.ShapeDtypeStruct((2, u, 16), jnp.float32)),
        mesh=_VMESH,
        compiler_params=_SC_PARAMS,
        scratch_types=[
            pltpu.VMEM_SHARED((u, D), jnp.float32),
            pltpu.VMEM_SHARED((u, 16), jnp.float32),
            pltpu.VMEM((CH // 2, 16), jnp.float32),
            pltpu.VMEM((rows_per_sub, D), jnp.float32),
            pltpu.VMEM((rows_per_sub, 16), jnp.float32),
        ],
    )
    def sc_kernel(hh2_hbm, idx_hbm, osum_hbm, ocnt_hbm,
                  acc_sum, acc_cnt, ones_v, zero_v, zcnt_v):
        cid = lax.axis_index("core")
        sid = lax.axis_index("subcore")

        @pl.loop(0, CH // 2)
        def _(i):
            ones_v.at[pl.ds(i, 1), :][...] = jnp.ones((1, 16), jnp.float32)

        @pl.loop(0, rows_per_sub)
        def _(i):
            for j in range(D // 16):
                zero_v.at[pl.ds(i, 1), pl.ds(j * 16, 16)][...] = (
                    jnp.zeros((1, 16), jnp.float32))
            zcnt_v.at[pl.ds(i, 1), :][...] = jnp.zeros((1, 16), jnp.float32)

        sl = pl.ds(sid * rows_per_sub, rows_per_sub)
        pltpu.sync_copy(zero_v, acc_sum.at[sl])
        pltpu.sync_copy(zcnt_v, acc_cnt.at[sl])
        plsc.subcore_barrier()

        n_chunks = h // CH

        def body(xl_vmem, xr_vmem, il_vmem, ir_vmem):
            pltpu.sync_copy(xl_vmem, acc_sum.at[il_vmem.at[0]], add=True)
            pltpu.sync_copy(xr_vmem, acc_sum.at[ir_vmem.at[0]], add=True)
            pltpu.sync_copy(ones_v, acc_cnt.at[il_vmem.at[0]], add=True)
            pltpu.sync_copy(ones_v, acc_cnt.at[ir_vmem.at[0]], add=True)

        pltpu.emit_pipeline(
            body,
            grid=(n_chunks,),
            in_specs=[
                pl.BlockSpec((CH // 2, D), lambda i: (i, 0)),
                pl.BlockSpec((CH // 2, D), lambda i: (i, 1)),
                pl.BlockSpec((1, CH // 2), lambda i: (0, i)),
                pl.BlockSpec((1, CH // 2), lambda i, n=n_chunks: (0, i + n)),
            ],
            out_specs=[],
            core_axis_name=("core", "subcore"),
            dimension_semantics=(pltpu.PARALLEL,),
        )(hh2_hbm, hh2_hbm, idx_hbm, idx_hbm)

        plsc.subcore_barrier()
        @pl.when(sid < 8)
        def _():
            pltpu.sync_copy(acc_sum.at[sl],
                            osum_hbm.at[cid, sl, pl.ds(0, D)])

        @pl.when(sid >= 8)
        def _():
            pltpu.sync_copy(
                acc_sum.at[sl],
                osum_hbm.at[cid, pl.ds((sid - 8) * rows_per_sub,
                                       rows_per_sub), pl.ds(D, D)])

        pltpu.sync_copy(acc_cnt.at[sl], ocnt_hbm.at[cid, sl])

    return sc_kernel(hh2, idx2d)


def _sc_gather(table, idx2d):
    n = idx2d.shape[1]

    @pl.kernel(
        out_type=jax.ShapeDtypeStruct((n, 2 * D), jnp.float32),
        mesh=_VMESH,
    )
    def sc_kernel(tab_hbm, i_hbm, o_hbm):
        def body(i_vmem, o_vmem):
            pltpu.sync_copy(tab_hbm.at[i_vmem.at[0]], o_vmem)

        pltpu.emit_pipeline(
            body,
            grid=(n // CH,),
            in_specs=[pl.BlockSpec((1, CH), lambda i: (0, i))],
            out_specs=[pl.BlockSpec((CH, 2 * D), lambda i: (i, 0))],
            core_axis_name=("core", "subcore"),
            dimension_semantics=(pltpu.PARALLEL,),
        )(i_hbm, o_hbm)

    return sc_kernel(table, idx2d)


def kernel(target_x, hist_x, hist_card_local_idx, target_card_local_idx,
           card_dense_feats, W1, b1, W2, b2, W3, b3, W4, b4, W5, b5, W6, b6):
    b = target_x.shape[0]
    u = card_dense_feats.shape[0]
    h = hist_x.shape[0]

    h2 = h // 2
    idx = hist_card_local_idx
    hh2_a = _encode_packed(hist_x, W1, b1, W2, b2, blk=8000,
                           n_rows=h2, row_off=0)
    hh2_b = _encode_packed(hist_x, W1, b1, W2, b2, blk=8000,
                           n_rows=h2, row_off=h2)

    up = ((u + 127) // 128) * 128
    psum_a, pcnt_a = _sc_segment_sum(hh2_a, idx[:h2].reshape(1, h2), up)
    psum_b, pcnt_b = _sc_segment_sum(hh2_b, idx[h2:].reshape(1, h2), up)

    dense_p = jnp.pad(card_dense_feats, ((0, up - u), (0, 0)))
    card_h = _card_mlp(psum_a, pcnt_a, psum_b, pcnt_b, dense_p,
                       W3[:CF], W3[CF:], b3, W4, b4)

    bp = ((b + CH - 1) // CH) * CH
    tidx = jnp.pad(target_card_local_idx, (0, bp - b)).reshape(1, bp)
    tch = _sc_gather(card_h, tidx)

    return _head(target_x, tch, W1, b1, W2, b2, W5, b5, W6, b6, blk=4096)
